# Initial kernel scaffold; baseline (speedup 1.0000x reference)
import jax
import jax.numpy as jnp
from jax.experimental import pallas as pl


def kernel(input_ids, embed_weight, proj_weight, proj_bias):
    raise NotImplementedError("write your pallas kernel here")


# speedup vs baseline: 1.0594x; 1.0594x over previous
import jax
import jax.numpy as jnp
from jax import lax
from jax.experimental import pallas as pl
from jax.experimental.pallas import tpu as pltpu
from jax.experimental.pallas import tpu_sc as plsc

_VOCAB = 12
_NC = 2
_NS = 16
_NW = _NC * _NS
_CHUNK = 128


_TROW = 16


def _table_body(e_ref, p_ref, b_ref, t_ref):
    t = lax.dot_general(
        e_ref[...], p_ref[...], (((1,), (1,)), ((), ())),
        preferred_element_type=jnp.float32,
    )
    t_ref[...] = jnp.concatenate(
        [t + b_ref[...], jnp.zeros((_VOCAB, _TROW - _VOCAB), jnp.float32)],
        axis=1,
    )


def _fused_table(embed_weight, proj_weight, proj_bias):
    return pl.pallas_call(
        _table_body,
        out_shape=jax.ShapeDtypeStruct((_VOCAB, _TROW), jnp.float32),
    )(embed_weight, proj_weight, proj_bias.reshape(1, _VOCAB))


def _make_lookup(n_tokens):
    per_w = n_tokens // _NW
    nchunks = per_w // _CHUNK
    assert per_w * _NW == n_tokens and nchunks * _CHUNK == per_w

    mesh = plsc.VectorSubcoreMesh(
        core_axis_name="c", subcore_axis_name="s",
        num_cores=_NC, num_subcores=_NS,
    )

    def body(ids_hbm, tab_hbm, out_hbm, idx_v, rows_v, sem):
        wid = lax.axis_index("s") * _NC + lax.axis_index("c")

        def step(c, carry):
            base = wid * per_w + c * _CHUNK
            pltpu.sync_copy(ids_hbm.at[pl.ds(base, _CHUNK)], idx_v)
            pltpu.async_copy(tab_hbm.at[idx_v], rows_v, sem).wait()
            pltpu.sync_copy(rows_v, out_hbm.at[pl.ds(base, _CHUNK)])
            return carry

        lax.fori_loop(0, nchunks, step, 0)

    return pl.kernel(
        body,
        out_type=jax.ShapeDtypeStruct((n_tokens, _TROW), jnp.float32),
        mesh=mesh,
        scratch_types=[
            pltpu.VMEM((_CHUNK,), jnp.int32),
            pltpu.VMEM((_CHUNK, _TROW), jnp.float32),
            pltpu.SemaphoreType.DMA,
        ],
        compiler_params=pltpu.CompilerParams(use_tc_tiling_on_sc=False),
    )


@jax.jit
def kernel(input_ids, embed_weight, proj_weight, proj_bias):
    b, s = input_ids.shape
    n = b * s
    tab = _fused_table(embed_weight, proj_weight, proj_bias)
    out = _make_lookup(n)(input_ids.reshape(n), tab)
    return out[:, :_VOCAB].reshape(b, s, _VOCAB)
